# Initial kernel scaffold; baseline (speedup 1.0000x reference)
import jax
import jax.numpy as jnp
from jax.experimental import pallas as pl


def kernel(key_xyz, key_features, query_xyz):
    raise NotImplementedError("write your pallas kernel here")


# speedup vs baseline: 1.8137x; 1.8137x over previous
import functools

import jax
import jax.numpy as jnp
import numpy as np
from jax import lax
from jax.experimental import pallas as pl

_RADIUS = np.float32(0.2)
_RAD2 = np.float32(0.2 * 0.2)
_NSAMPLE = 32
_SBLK = 8
_CCHUNK = 512


def _sg_kernel(xyzt_ref, q_ref, feat_ref, gx_ref, gf_ref):
    n = xyzt_ref.shape[2]
    ns = _NSAMPLE

    xyzt = xyzt_ref[0]
    q = q_ref[0]

    x2 = (xyzt[0:1, :] * xyzt[0:1, :]
          + xyzt[1:2, :] * xyzt[1:2, :]
          + xyzt[2:3, :] * xyzt[2:3, :])
    q2 = jnp.sum(q * q, axis=1, keepdims=True)
    qx = jnp.dot(q, xyzt, preferred_element_type=jnp.float32)
    d2 = q2 + x2 - 2.0 * qx
    mask = d2 < _RAD2

    mbf = mask.astype(jnp.bfloat16)
    row_i = lax.broadcasted_iota(jnp.int32, (_CCHUNK, _CCHUNK), 0)
    col_j = lax.broadcasted_iota(jnp.int32, (_CCHUNK, _CCHUNK), 1)
    ltri = (row_i <= col_j).astype(jnp.bfloat16)
    carry = jnp.zeros((_SBLK, 1), jnp.float32)
    chunks = []
    for j in range(n // _CCHUNK):
        mc = mbf[:, j * _CCHUNK:(j + 1) * _CCHUNK]
        cc = jnp.dot(mc, ltri, preferred_element_type=jnp.float32) + carry
        chunks.append(cc)
        carry = cc[:, _CCHUNK - 1:_CCHUNK]
    c = jnp.concatenate(chunks, axis=1)
    total = carry

    rows = _SBLK * ns
    c_e = jnp.broadcast_to(c[:, None, :], (_SBLK, ns, n)).reshape(rows, n)
    m_e = jnp.broadcast_to(mask[:, None, :], (_SBLK, ns, n)).reshape(rows, n)
    t_e = jnp.broadcast_to(total[:, None, :], (_SBLK, ns, 1)).reshape(rows, 1)
    kk1 = (lax.broadcasted_iota(jnp.int32, (_SBLK, ns, 1), 1)
           .reshape(rows, 1).astype(jnp.float32) + 1.0)
    iota_n = lax.broadcasted_iota(jnp.int32, (1, n), 1)
    sel = (m_e & ((c_e == kk1) | ((c_e == 1.0) & (kk1 > t_e)))) \
        | ((t_e == 0.0) & (iota_n == 0))
    onehot = sel.astype(jnp.float32)

    dn = (((1,), (1,)), ((), ()))
    gf = lax.dot_general(feat_ref[0], onehot, dn,
                         precision=lax.Precision.HIGHEST,
                         preferred_element_type=jnp.float32)
    gx = lax.dot_general(xyzt, onehot, dn,
                         precision=lax.Precision.HIGHEST,
                         preferred_element_type=jnp.float32)
    q_e = jnp.broadcast_to(q.T[:, :, None], (3, _SBLK, ns)).reshape(3, rows)
    gx_ref[0] = (gx - q_e) / _RADIUS
    gf_ref[0] = gf


@jax.jit
def kernel(key_xyz, key_features, query_xyz):
    b, n, _ = key_xyz.shape
    c = key_features.shape[1]
    s = query_xyz.shape[1]
    ns = _NSAMPLE
    sblocks = s // _SBLK

    xyzt = jnp.transpose(key_xyz, (0, 2, 1))
    q = query_xyz[:, :, :3]

    grid = (b, sblocks)
    gx, gf = pl.pallas_call(
        _sg_kernel,
        grid=grid,
        in_specs=[
            pl.BlockSpec((1, 3, n), lambda i, j: (i, 0, 0)),
            pl.BlockSpec((1, _SBLK, 3), lambda i, j: (i, j, 0)),
            pl.BlockSpec((1, c, n), lambda i, j: (i, 0, 0)),
        ],
        out_specs=[
            pl.BlockSpec((1, 3, _SBLK * ns), lambda i, j: (i, 0, j)),
            pl.BlockSpec((1, c, _SBLK * ns), lambda i, j: (i, 0, j)),
        ],
        out_shape=[
            jax.ShapeDtypeStruct((b, 3, s * ns), jnp.float32),
            jax.ShapeDtypeStruct((b, c, s * ns), jnp.float32),
        ],
    )(xyzt, q, key_features)

    grouped_xyz = gx.reshape(b, 3, s, ns)
    grouped_features = gf.reshape(b, c, s, ns)
    return grouped_xyz, grouped_features


# speedup vs baseline: 4.0694x; 2.2438x over previous
import functools

import jax
import jax.numpy as jnp
import numpy as np
from jax import lax
from jax.experimental import pallas as pl
from jax.experimental.pallas import tpu as pltpu
from jax.experimental.pallas import tpu_sc as plsc

_RADIUS = np.float32(0.2)
_RAD2 = np.float32(0.2 * 0.2)
_NSAMPLE = 32
_SBLK = 8
_CCHUNK = 512

_B, _N, _S, _C = 8, 4096, 1024, 128
_NW = 32
_CG = _C // 4
_SUB = 8
_SCH = 2048


def _idx_kernel(xyzt_ref, q_ref, idx_ref):
    n = xyzt_ref.shape[2]
    ns = _NSAMPLE

    xyzt = xyzt_ref[0]
    q = q_ref[0]

    x2 = (xyzt[0:1, :] * xyzt[0:1, :]
          + xyzt[1:2, :] * xyzt[1:2, :]
          + xyzt[2:3, :] * xyzt[2:3, :])
    q2 = jnp.sum(q * q, axis=1, keepdims=True)
    qx = jnp.dot(q, xyzt, preferred_element_type=jnp.float32)
    d2 = q2 + x2 - 2.0 * qx
    mask = d2 < _RAD2

    mbf = mask.astype(jnp.bfloat16)
    row_i = lax.broadcasted_iota(jnp.int32, (_CCHUNK, _CCHUNK), 0)
    col_j = lax.broadcasted_iota(jnp.int32, (_CCHUNK, _CCHUNK), 1)
    ltri = (row_i <= col_j).astype(jnp.bfloat16)
    carry = jnp.zeros((_SBLK, 1), jnp.float32)
    chunks = []
    for j in range(n // _CCHUNK):
        mc = mbf[:, j * _CCHUNK:(j + 1) * _CCHUNK]
        cc = jnp.dot(mc, ltri, preferred_element_type=jnp.float32) + carry
        chunks.append(cc)
        carry = cc[:, _CCHUNK - 1:_CCHUNK]
    c = jnp.concatenate(chunks, axis=1)
    total = carry

    rows = _SBLK * ns
    c_e = jnp.broadcast_to(c[:, None, :], (_SBLK, ns, n)).reshape(rows, n)
    m_e = jnp.broadcast_to(mask[:, None, :], (_SBLK, ns, n)).reshape(rows, n)
    t_e = jnp.broadcast_to(total[:, None, :], (_SBLK, ns, 1)).reshape(rows, 1)
    kk1 = (lax.broadcasted_iota(jnp.int32, (_SBLK, ns, 1), 1)
           .reshape(rows, 1).astype(jnp.float32) + 1.0)
    iota_n = lax.broadcasted_iota(jnp.int32, (1, n), 1)
    sel = (m_e & ((c_e == kk1) | ((c_e == 1.0) & (kk1 > t_e)))) \
        | ((t_e == 0.0) & (iota_n == 0))
    idxf = jnp.sum(jnp.where(sel, iota_n.astype(jnp.float32), 0.0), axis=1)
    idx_ref[0] = idxf.astype(jnp.int32).reshape(_SBLK, ns)


def _ball_query_idx(xyzt, q):
    grid = (_B, _S // _SBLK)
    return pl.pallas_call(
        _idx_kernel,
        grid=grid,
        in_specs=[
            pl.BlockSpec((1, 3, _N), lambda i, j: (i, 0, 0)),
            pl.BlockSpec((1, _SBLK, 3), lambda i, j: (i, j, 0)),
        ],
        out_specs=pl.BlockSpec((1, _SBLK, _NSAMPLE), lambda i, j: (i, j, 0)),
        out_shape=jax.ShapeDtypeStruct((_B, _S, _NSAMPLE), jnp.int32),
    )(xyzt, q)


def _sc_gather_body(feat_hbm, idx_hbm, xyzt_hbm, qt_hbm,
                    gf_hbm, gx_hbm,
                    idx_v, tabs_v, stage_v, xtab_v, xstage_v, q_v, sem):
    ns = _NSAMPLE
    wid = lax.axis_index("s") * 2 + lax.axis_index("c")
    b = wid // 4
    g = wid % 4

    pltpu.sync_copy(idx_hbm.at[b], idx_v)

    def do_rows(c0, tab_rows, n_rows):
        def sc_loop(sc, _):
            def g_loop(gi, _):
                off = sc * _SCH + gi * 16
                iv = idx_v[pl.ds(off, 16)]
                for r in range(n_rows):
                    rv = jnp.full((16,), r, jnp.int32)
                    stage_v[r, pl.ds(gi * 16, 16)] = plsc.load_gather(
                        tab_rows, [rv, iv])
                return 0
            lax.fori_loop(0, _SCH // 16, g_loop, 0, unroll=2)
            cps = []
            for r in range(n_rows):
                cps.append(pltpu.async_copy(
                    stage_v.at[r],
                    gf_hbm.at[b, c0 + r, pl.ds(sc * _SCH, _SCH)], sem))
            for cp in cps:
                cp.wait()
            return 0
        lax.fori_loop(0, (_S * ns) // _SCH, sc_loop, 0)

    for sub in range(_CG // _SUB):
        c0 = g * _CG + sub * _SUB
        for r in range(_SUB):
            pltpu.sync_copy(feat_hbm.at[b, c0 + r], tabs_v.at[r])
        do_rows(c0, tabs_v, _SUB)

    @pl.when(g < 3)
    def _xyz():
        pltpu.sync_copy(xyzt_hbm.at[b, g], xtab_v)
        pltpu.sync_copy(qt_hbm.at[b, g], q_v)

        def sc_loop(sc, _):
            def g_loop(gi, _):
                off = sc * _SCH + gi * 16
                iv = idx_v[pl.ds(off, 16)]
                vals = plsc.load_gather(xtab_v, [iv])
                si = lax.shift_right_logical(lax.iota(jnp.int32, 16) + off, 5)
                qs = plsc.load_gather(q_v, [si])
                xstage_v[pl.ds(gi * 16, 16)] = (vals - qs) / _RADIUS
                return 0
            lax.fori_loop(0, _SCH // 16, g_loop, 0, unroll=2)
            pltpu.async_copy(
                xstage_v, gx_hbm.at[b, g, pl.ds(sc * _SCH, _SCH)], sem
            ).wait()
            return 0
        lax.fori_loop(0, (_S * ns) // _SCH, sc_loop, 0)


def _sc_gather(key_features, idx, xyzt, qt):
    mesh = plsc.VectorSubcoreMesh(core_axis_name="c", subcore_axis_name="s")
    fn = functools.partial(
        pl.kernel, mesh=mesh,
        compiler_params=pltpu.CompilerParams(needs_layout_passes=False),
        out_type=[
            jax.ShapeDtypeStruct((_B, _C, _S * _NSAMPLE), jnp.float32),
            jax.ShapeDtypeStruct((_B, 3, _S * _NSAMPLE), jnp.float32),
        ],
        scratch_types=[
            pltpu.VMEM((_S * _NSAMPLE,), jnp.int32),
            pltpu.VMEM((_SUB, _N), jnp.float32),
            pltpu.VMEM((_SUB, _SCH), jnp.float32),
            pltpu.VMEM((_N,), jnp.float32),
            pltpu.VMEM((_SCH,), jnp.float32),
            pltpu.VMEM((_S,), jnp.float32),
            pltpu.SemaphoreType.DMA,
        ],
    )(_sc_gather_body)
    return fn(key_features, idx.reshape(_B, _S * _NSAMPLE), xyzt, qt)


@jax.jit
def kernel(key_xyz, key_features, query_xyz):
    xyzt = jnp.transpose(key_xyz, (0, 2, 1))
    q = query_xyz[:, :, :3]
    qt = jnp.transpose(q, (0, 2, 1))

    idx = _ball_query_idx(xyzt, q)
    gf, gx = _sc_gather(key_features, idx, xyzt, qt)

    grouped_xyz = gx.reshape(_B, 3, _S, _NSAMPLE)
    grouped_features = gf.reshape(_B, _C, _S, _NSAMPLE)
    return grouped_xyz, grouped_features


# speedup vs baseline: 11.9266x; 2.9308x over previous
import functools

import jax
import jax.numpy as jnp
import numpy as np
from jax import lax
from jax.experimental import pallas as pl
from jax.experimental.pallas import tpu as pltpu
from jax.experimental.pallas import tpu_sc as plsc

_RADIUS = np.float32(0.2)
_RAD2 = np.float32(0.2 * 0.2)
_NSAMPLE = 32
_SBLK = 128
_B, _N, _S, _C = 8, 4096, 1024, 128
_W = _N // 16

_CG = _C // 4
_SUB = 8
_SCH = 2048

_QW = (_S * _B) // 32
_QCH = 16


def _mask_kernel(xyzt_ref, q_ref, pow2_ref, pk_ref):
    xyzt = xyzt_ref[0]
    q = q_ref[0]

    x2 = (xyzt[0:1, :] * xyzt[0:1, :]
          + xyzt[1:2, :] * xyzt[1:2, :]
          + xyzt[2:3, :] * xyzt[2:3, :])
    q2 = jnp.sum(q * q, axis=1, keepdims=True)
    qx = jnp.dot(q, xyzt, preferred_element_type=jnp.float32)
    d2 = q2 + x2 - 2.0 * qx
    mask = d2 < _RAD2

    packed = jnp.dot(mask.astype(jnp.bfloat16), pow2_ref[...],
                     preferred_element_type=jnp.float32)
    pk_ref[0] = packed.astype(jnp.int32)


def _ball_mask_packed(xyzt, q, pow2):
    grid = (_B, _S // _SBLK)
    return pl.pallas_call(
        _mask_kernel,
        grid=grid,
        in_specs=[
            pl.BlockSpec((1, 3, _N), lambda i, j: (i, 0, 0)),
            pl.BlockSpec((1, _SBLK, 3), lambda i, j: (i, j, 0)),
            pl.BlockSpec((_N, _W), lambda i, j: (0, 0)),
        ],
        out_specs=pl.BlockSpec((1, _SBLK, _W), lambda i, j: (i, j, 0)),
        out_shape=jax.ShapeDtypeStruct((_B, _S, _W), jnp.int32),
    )(xyzt, q, pow2)


def _sc_select_body(pk_hbm, idx_hbm, pw_v, buf_v, out_v, sem):
    ns = _NSAMPLE
    wid = lax.axis_index("s") * 2 + lax.axis_index("c")
    q0 = wid * _QW
    lanes = lax.iota(jnp.int32, 16)

    def chunk_loop(ch, _):
        qbase = q0 + ch * _QCH
        pltpu.sync_copy(pk_hbm.at[pl.ds(qbase * _W, _QCH * _W)], pw_v)

        def q_loop(qi, _):
            def cond(carry):
                w, cnt = carry
                return (cnt < ns) & (w < _W)

            def body(carry):
                w, cnt = carry
                iw = jnp.zeros((16,), jnp.int32) + (qi * _W + w)
                wv = plsc.load_gather(pw_v, [iw])
                bits = lax.shift_right_logical(wv, lanes) & 1
                m = bits != 0
                ivec = lanes + w * 16
                plsc.store_compressed(buf_v.at[pl.ds(cnt, 16)], ivec, mask=m)
                cntv = plsc.all_reduce_population_count(m)
                return w + 1, cnt + cntv[0]

            _, cnt = lax.while_loop(cond, body, (0, 0))

            v0 = buf_v[pl.ds(0, 16)]
            padv = jnp.where(cnt > 0, jnp.zeros((16,), jnp.int32) + v0[0], 0)
            for h in range(2):
                vals = buf_v[pl.ds(h * 16, 16)]
                pos = lanes + h * 16
                out_v[pl.ds(qi * ns + h * 16, 16)] = jnp.where(
                    pos < cnt, vals, padv)
            return 0

        lax.fori_loop(0, _QCH, q_loop, 0)
        pltpu.async_copy(
            out_v, idx_hbm.at[pl.ds(qbase * ns, _QCH * ns)], sem).wait()
        return 0

    lax.fori_loop(0, _QW // _QCH, chunk_loop, 0)


def _sc_select(packed):
    mesh = plsc.VectorSubcoreMesh(core_axis_name="c", subcore_axis_name="s")
    fn = functools.partial(
        pl.kernel, mesh=mesh,
        compiler_params=pltpu.CompilerParams(needs_layout_passes=False),
        out_type=jax.ShapeDtypeStruct((_B * _S * _NSAMPLE,), jnp.int32),
        scratch_types=[
            pltpu.VMEM((_QCH * _W,), jnp.int32),
            pltpu.VMEM((64,), jnp.int32),
            pltpu.VMEM((_QCH * _NSAMPLE,), jnp.int32),
            pltpu.SemaphoreType.DMA,
        ],
    )(_sc_select_body)
    return fn(packed.reshape(_B * _S * _W))


def _sc_gather_body(feat_hbm, idx_hbm, xyzt_hbm, qt_hbm,
                    gf_hbm, gx_hbm,
                    idx_v, tabs_v, stage_v, xtab_v, xstage_v, q_v, sem):
    ns = _NSAMPLE
    wid = lax.axis_index("s") * 2 + lax.axis_index("c")
    b = wid // 4
    g = wid % 4

    pltpu.sync_copy(idx_hbm.at[b], idx_v)

    def do_rows(c0, tab_rows, n_rows):
        def sc_loop(sc, _):
            def g_loop(gi, _):
                off = sc * _SCH + gi * 16
                iv = idx_v[pl.ds(off, 16)]
                for r in range(n_rows):
                    rv = jnp.full((16,), r, jnp.int32)
                    stage_v[r, pl.ds(gi * 16, 16)] = plsc.load_gather(
                        tab_rows, [rv, iv])
                return 0
            lax.fori_loop(0, _SCH // 16, g_loop, 0, unroll=2)
            cps = []
            for r in range(n_rows):
                cps.append(pltpu.async_copy(
                    stage_v.at[r],
                    gf_hbm.at[b, c0 + r, pl.ds(sc * _SCH, _SCH)], sem))
            for cp in cps:
                cp.wait()
            return 0
        lax.fori_loop(0, (_S * ns) // _SCH, sc_loop, 0)

    for sub in range(_CG // _SUB):
        c0 = g * _CG + sub * _SUB
        for r in range(_SUB):
            pltpu.sync_copy(feat_hbm.at[b, c0 + r], tabs_v.at[r])
        do_rows(c0, tabs_v, _SUB)

    @pl.when(g < 3)
    def _xyz():
        pltpu.sync_copy(xyzt_hbm.at[b, g], xtab_v)
        pltpu.sync_copy(qt_hbm.at[b, g], q_v)

        def sc_loop(sc, _):
            def g_loop(gi, _):
                off = sc * _SCH + gi * 16
                iv = idx_v[pl.ds(off, 16)]
                vals = plsc.load_gather(xtab_v, [iv])
                si = lax.shift_right_logical(lax.iota(jnp.int32, 16) + off, 5)
                qs = plsc.load_gather(q_v, [si])
                xstage_v[pl.ds(gi * 16, 16)] = (vals - qs) / _RADIUS
                return 0
            lax.fori_loop(0, _SCH // 16, g_loop, 0, unroll=2)
            pltpu.async_copy(
                xstage_v, gx_hbm.at[b, g, pl.ds(sc * _SCH, _SCH)], sem
            ).wait()
            return 0
        lax.fori_loop(0, (_S * ns) // _SCH, sc_loop, 0)


def _sc_gather(key_features, idx, xyzt, qt):
    mesh = plsc.VectorSubcoreMesh(core_axis_name="c", subcore_axis_name="s")
    fn = functools.partial(
        pl.kernel, mesh=mesh,
        compiler_params=pltpu.CompilerParams(needs_layout_passes=False),
        out_type=[
            jax.ShapeDtypeStruct((_B, _C, _S * _NSAMPLE), jnp.float32),
            jax.ShapeDtypeStruct((_B, 3, _S * _NSAMPLE), jnp.float32),
        ],
        scratch_types=[
            pltpu.VMEM((_S * _NSAMPLE,), jnp.int32),
            pltpu.VMEM((_SUB, _N), jnp.float32),
            pltpu.VMEM((_SUB, _SCH), jnp.float32),
            pltpu.VMEM((_N,), jnp.float32),
            pltpu.VMEM((_SCH,), jnp.float32),
            pltpu.VMEM((_S,), jnp.float32),
            pltpu.SemaphoreType.DMA,
        ],
    )(_sc_gather_body)
    return fn(key_features, idx.reshape(_B, _S * _NSAMPLE), xyzt, qt)


@jax.jit
def kernel(key_xyz, key_features, query_xyz):
    xyzt = jnp.transpose(key_xyz, (0, 2, 1))
    q = query_xyz[:, :, :3]
    qt = jnp.transpose(q, (0, 2, 1))

    nn = jnp.arange(_N, dtype=jnp.int32)
    pw = (1 << (nn % 16)).astype(jnp.float32)
    pow2 = jnp.where((nn[:, None] // 16)
                     == jnp.arange(_W, dtype=jnp.int32)[None, :],
                     pw[:, None], 0.0).astype(jnp.bfloat16)

    packed = _ball_mask_packed(xyzt, q, pow2)
    idx = _sc_select(packed)
    gf, gx = _sc_gather(key_features, idx, xyzt, qt)

    grouped_xyz = gx.reshape(_B, 3, _S, _NSAMPLE)
    grouped_features = gf.reshape(_B, _C, _S, _NSAMPLE)
    return grouped_xyz, grouped_features


# speedup vs baseline: 12.1767x; 1.0210x over previous
import functools

import jax
import jax.numpy as jnp
import numpy as np
from jax import lax
from jax.experimental import pallas as pl
from jax.experimental.pallas import tpu as pltpu
from jax.experimental.pallas import tpu_sc as plsc

_RADIUS = np.float32(0.2)
_RAD2 = np.float32(0.2 * 0.2)
_NSAMPLE = 32
_SBLK = 128
_B, _N, _S, _C = 8, 4096, 1024, 128
_W = _N // 16

_CG = _C // 4
_SUB = 8
_SCH = 2048

_QW = (_S * _B) // 32
_QCH = 16


def _mask_kernel(xyzt_ref, q_ref, pow2_ref, pk_ref):
    xyzt = xyzt_ref[0]
    q = q_ref[0]

    x2 = (xyzt[0:1, :] * xyzt[0:1, :]
          + xyzt[1:2, :] * xyzt[1:2, :]
          + xyzt[2:3, :] * xyzt[2:3, :])
    q2 = jnp.sum(q * q, axis=1, keepdims=True)
    qx = jnp.dot(q, xyzt, preferred_element_type=jnp.float32)
    d2 = q2 + x2 - 2.0 * qx
    mask = d2 < _RAD2

    packed = jnp.dot(mask.astype(jnp.bfloat16), pow2_ref[...],
                     preferred_element_type=jnp.float32)
    pk_ref[0] = packed.astype(jnp.int32)


def _ball_mask_packed(xyzt, q, pow2):
    grid = (_B, _S // _SBLK)
    return pl.pallas_call(
        _mask_kernel,
        grid=grid,
        in_specs=[
            pl.BlockSpec((1, 3, _N), lambda i, j: (i, 0, 0)),
            pl.BlockSpec((1, _SBLK, 3), lambda i, j: (i, j, 0)),
            pl.BlockSpec((_N, _W), lambda i, j: (0, 0)),
        ],
        out_specs=pl.BlockSpec((1, _SBLK, _W), lambda i, j: (i, j, 0)),
        out_shape=jax.ShapeDtypeStruct((_B, _S, _W), jnp.int32),
    )(xyzt, q, pow2)


def _sc_select_body(pk_hbm, idx_hbm, pw_v, buf_v, out_v, sem):
    ns = _NSAMPLE
    wid = lax.axis_index("s") * 2 + lax.axis_index("c")
    q0 = wid * _QW
    lanes = lax.iota(jnp.int32, 16)

    def chunk_loop(ch, _):
        qbase = q0 + ch * _QCH
        pltpu.sync_copy(pk_hbm.at[pl.ds(qbase * _W, _QCH * _W)], pw_v)

        def q_loop(qi, _):
            def cond(carry):
                w, cnt = carry
                return (cnt < ns) & (w < _W)

            def body(carry):
                w, cnt = carry
                iw = jnp.zeros((16,), jnp.int32) + (qi * _W + w)
                wv = plsc.load_gather(pw_v, [iw])
                bits = lax.shift_right_logical(wv, lanes) & 1
                m = bits != 0
                ivec = lanes + w * 16
                plsc.store_compressed(buf_v.at[pl.ds(cnt, 16)], ivec, mask=m)
                cntv = plsc.all_reduce_population_count(m)
                return w + 1, cnt + cntv[0]

            _, cnt = lax.while_loop(cond, body, (0, 0))

            v0 = buf_v[pl.ds(0, 16)]
            padv = jnp.where(cnt > 0, jnp.zeros((16,), jnp.int32) + v0[0], 0)
            for h in range(2):
                vals = buf_v[pl.ds(h * 16, 16)]
                pos = lanes + h * 16
                out_v[pl.ds(qi * ns + h * 16, 16)] = jnp.where(
                    pos < cnt, vals, padv)
            return 0

        lax.fori_loop(0, _QCH, q_loop, 0)
        pltpu.async_copy(
            out_v, idx_hbm.at[pl.ds(qbase * ns, _QCH * ns)], sem).wait()
        return 0

    lax.fori_loop(0, _QW // _QCH, chunk_loop, 0)


def _sc_select(packed):
    mesh = plsc.VectorSubcoreMesh(core_axis_name="c", subcore_axis_name="s")
    fn = functools.partial(
        pl.kernel, mesh=mesh,
        compiler_params=pltpu.CompilerParams(needs_layout_passes=False),
        out_type=jax.ShapeDtypeStruct((_B * _S * _NSAMPLE,), jnp.int32),
        scratch_types=[
            pltpu.VMEM((_QCH * _W,), jnp.int32),
            pltpu.VMEM((64,), jnp.int32),
            pltpu.VMEM((_QCH * _NSAMPLE,), jnp.int32),
            pltpu.SemaphoreType.DMA,
        ],
    )(_sc_select_body)
    return fn(packed.reshape(_B * _S * _W))


def _sc_gather_body(feat_hbm, idx_hbm, xyzt_hbm, qt_hbm,
                    gf_hbm, gx_hbm,
                    idx_v, tabs_v, stage_v, xtab_v, xstage_v, q_v, sem):
    ns = _NSAMPLE
    nsc = (_S * ns) // _SCH
    wid = lax.axis_index("s") * 2 + lax.axis_index("c")
    b = wid // 4
    g = wid % 4

    pltpu.sync_copy(idx_hbm.at[pl.ds(b * (_S * ns), _S * ns)], idx_v)

    def do_rows(c0, tab_rows, n_rows):
        def desc(sc, buf):
            return pltpu.make_async_copy(
                stage_v.at[buf],
                gf_hbm.at[b, pl.ds(c0, n_rows), pl.ds(sc * _SCH, _SCH)],
                sem)

        def sc_loop(sc, _):
            buf = lax.rem(sc, 2)

            @pl.when(sc >= 2)
            def _drain():
                desc(sc, buf).wait()

            def g_loop(gi, _):
                off = sc * _SCH + gi * 16
                iv = idx_v[pl.ds(off, 16)]
                for r in range(n_rows):
                    rv = jnp.full((16,), r, jnp.int32)
                    stage_v[buf, r, pl.ds(gi * 16, 16)] = plsc.load_gather(
                        tab_rows, [rv, iv])
                return 0
            lax.fori_loop(0, _SCH // 16, g_loop, 0, unroll=2)
            desc(sc, buf).start()
            return 0
        lax.fori_loop(0, nsc, sc_loop, 0)
        for buf in range(2):
            desc(0, buf).wait()

    for sub in range(_CG // _SUB):
        c0 = g * _CG + sub * _SUB
        for r in range(_SUB):
            pltpu.sync_copy(
                feat_hbm.at[pl.ds((b * _C + c0 + r) * _N, _N)], tabs_v.at[r])
        do_rows(c0, tabs_v, _SUB)

    @pl.when(g < 3)
    def _xyz():
        pltpu.sync_copy(xyzt_hbm.at[pl.ds((b * 3 + g) * _N, _N)], xtab_v)
        pltpu.sync_copy(qt_hbm.at[pl.ds((b * 3 + g) * _S, _S)], q_v)

        def sc_loop(sc, _):
            def g_loop(gi, _):
                off = sc * _SCH + gi * 16
                iv = idx_v[pl.ds(off, 16)]
                vals = plsc.load_gather(xtab_v, [iv])
                si = lax.shift_right_logical(lax.iota(jnp.int32, 16) + off, 5)
                qs = plsc.load_gather(q_v, [si])
                xstage_v[pl.ds(gi * 16, 16)] = (vals - qs) / _RADIUS
                return 0
            lax.fori_loop(0, _SCH // 16, g_loop, 0, unroll=2)
            pltpu.async_copy(
                xstage_v, gx_hbm.at[b, g, pl.ds(sc * _SCH, _SCH)], sem
            ).wait()
            return 0
        lax.fori_loop(0, (_S * ns) // _SCH, sc_loop, 0)


def _sc_gather(key_features, idx, xyzt, qt):
    mesh = plsc.VectorSubcoreMesh(core_axis_name="c", subcore_axis_name="s")
    fn = functools.partial(
        pl.kernel, mesh=mesh,
        compiler_params=pltpu.CompilerParams(needs_layout_passes=False),
        out_type=[
            jax.ShapeDtypeStruct((_B, _C, _S * _NSAMPLE), jnp.float32),
            jax.ShapeDtypeStruct((_B, 3, _S * _NSAMPLE), jnp.float32),
        ],
        scratch_types=[
            pltpu.VMEM((_S * _NSAMPLE,), jnp.int32),
            pltpu.VMEM((_SUB, _N), jnp.float32),
            pltpu.VMEM((2, _SUB, _SCH), jnp.float32),
            pltpu.VMEM((_N,), jnp.float32),
            pltpu.VMEM((_SCH,), jnp.float32),
            pltpu.VMEM((_S,), jnp.float32),
            pltpu.SemaphoreType.DMA,
        ],
    )(_sc_gather_body)
    return fn(key_features.reshape(_B * _C * _N), idx,
              xyzt.reshape(_B * 3 * _N), qt.reshape(_B * 3 * _S))


@jax.jit
def kernel(key_xyz, key_features, query_xyz):
    xyzt = jnp.transpose(key_xyz, (0, 2, 1))
    q = query_xyz[:, :, :3]
    qt = jnp.transpose(q, (0, 2, 1))

    nn = jnp.arange(_N, dtype=jnp.int32)
    pw = (1 << (nn % 16)).astype(jnp.float32)
    pow2 = jnp.where((nn[:, None] // 16)
                     == jnp.arange(_W, dtype=jnp.int32)[None, :],
                     pw[:, None], 0.0).astype(jnp.bfloat16)

    packed = _ball_mask_packed(xyzt, q, pow2)
    idx = _sc_select(packed)
    gf, gx = _sc_gather(key_features, idx, xyzt, qt)

    grouped_xyz = gx.reshape(_B, 3, _S, _NSAMPLE)
    grouped_features = gf.reshape(_B, _C, _S, _NSAMPLE)
    return grouped_xyz, grouped_features


# speedup vs baseline: 15.7763x; 1.2956x over previous
import functools

import jax
import jax.numpy as jnp
import numpy as np
from jax import lax
from jax.experimental import pallas as pl
from jax.experimental.pallas import tpu as pltpu
from jax.experimental.pallas import tpu_sc as plsc

_RADIUS = np.float32(0.2)
_RAD2 = np.float32(0.2 * 0.2)
_NSAMPLE = 32
_SBLK = 128
_B, _N, _S, _C = 8, 4096, 1024, 128
_W = _N // 16

_CG = _C // 4
_SUB = 8
_SCH = 2048

_QW = (_S * _B) // 32
_QCH = 16


def _mask_kernel(xyzt_ref, q_ref, pow2_ref, feat_ref, pk_ref, featl_ref):
    @pl.when(pl.program_id(1) == 0)
    def _pass_feat():
        for cc in range(_C):
            featl_ref[pl.ds(cc * _N, _N)] = feat_ref[0, cc]

    xyzt = xyzt_ref[0]
    q = q_ref[0]

    x2 = (xyzt[0:1, :] * xyzt[0:1, :]
          + xyzt[1:2, :] * xyzt[1:2, :]
          + xyzt[2:3, :] * xyzt[2:3, :])
    q2 = jnp.sum(q * q, axis=1, keepdims=True)
    qx = jnp.dot(q, xyzt, preferred_element_type=jnp.float32)
    d2 = q2 + x2 - 2.0 * qx
    mask = d2 < _RAD2

    packed = jnp.dot(mask.astype(jnp.bfloat16), pow2_ref[...],
                     preferred_element_type=jnp.float32)
    pk_ref[0] = packed.astype(jnp.int32)


def _ball_mask_packed(xyzt, q, pow2, feat):
    grid = (_B, _S // _SBLK)
    return pl.pallas_call(
        _mask_kernel,
        grid=grid,
        in_specs=[
            pl.BlockSpec((1, 3, _N), lambda i, j: (i, 0, 0)),
            pl.BlockSpec((1, _SBLK, 3), lambda i, j: (i, j, 0)),
            pl.BlockSpec((_N, _W), lambda i, j: (0, 0)),
            pl.BlockSpec((1, _C, _N), lambda i, j: (i, 0, 0)),
        ],
        out_specs=[
            pl.BlockSpec((1, _SBLK, _W), lambda i, j: (i, j, 0)),
            pl.BlockSpec((_C * _N,), lambda i, j: (i,)),
        ],
        out_shape=[
            jax.ShapeDtypeStruct((_B, _S, _W), jnp.int32),
            jax.ShapeDtypeStruct((_B * _C * _N,), jnp.float32),
        ],
    )(xyzt, q, pow2, feat)


def _sc_select_body(pk_hbm, idx_hbm, pw_v, buf_v, out_v, sem):
    ns = _NSAMPLE
    wid = lax.axis_index("s") * 2 + lax.axis_index("c")
    q0 = wid * _QW
    lanes = lax.iota(jnp.int32, 16)

    def chunk_loop(ch, _):
        qbase = q0 + ch * _QCH
        pltpu.sync_copy(pk_hbm.at[pl.ds(qbase * _W, _QCH * _W)], pw_v)

        def q_loop(qi, _):
            def cond(carry):
                w, cnt = carry
                return (cnt < ns) & (w < _W)

            def body(carry):
                w, cnt = carry
                iw = jnp.zeros((16,), jnp.int32) + (qi * _W + w)
                wv = plsc.load_gather(pw_v, [iw])
                bits = lax.shift_right_logical(wv, lanes) & 1
                m = bits != 0
                ivec = lanes + w * 16
                plsc.store_compressed(buf_v.at[pl.ds(cnt, 16)], ivec, mask=m)
                cntv = plsc.all_reduce_population_count(m)
                return w + 1, cnt + cntv[0]

            _, cnt = lax.while_loop(cond, body, (0, 0))

            v0 = buf_v[pl.ds(0, 16)]
            padv = jnp.where(cnt > 0, jnp.zeros((16,), jnp.int32) + v0[0], 0)
            for h in range(2):
                vals = buf_v[pl.ds(h * 16, 16)]
                pos = lanes + h * 16
                out_v[pl.ds(qi * ns + h * 16, 16)] = jnp.where(
                    pos < cnt, vals, padv)
            return 0

        lax.fori_loop(0, _QCH, q_loop, 0)
        pltpu.async_copy(
            out_v, idx_hbm.at[pl.ds(qbase * ns, _QCH * ns)], sem).wait()
        return 0

    lax.fori_loop(0, _QW // _QCH, chunk_loop, 0)


def _sc_select(packed):
    mesh = plsc.VectorSubcoreMesh(core_axis_name="c", subcore_axis_name="s")
    fn = functools.partial(
        pl.kernel, mesh=mesh,
        compiler_params=pltpu.CompilerParams(needs_layout_passes=False),
        out_type=jax.ShapeDtypeStruct((_B * _S * _NSAMPLE,), jnp.int32),
        scratch_types=[
            pltpu.VMEM((_QCH * _W,), jnp.int32),
            pltpu.VMEM((64,), jnp.int32),
            pltpu.VMEM((_QCH * _NSAMPLE,), jnp.int32),
            pltpu.SemaphoreType.DMA,
        ],
    )(_sc_select_body)
    return fn(packed.reshape(_B * _S * _W))


def _sc_gather_body(feat_hbm, idx_hbm, xyzt_hbm, qt_hbm,
                    gf_hbm, gx_hbm,
                    idx_v, tabs_v, stage_v, xtab_v, xstage_v, q_v, sem):
    ns = _NSAMPLE
    nsc = (_S * ns) // _SCH
    wid = lax.axis_index("s") * 2 + lax.axis_index("c")
    b = wid // 4
    g = wid % 4

    pltpu.sync_copy(idx_hbm.at[pl.ds(b * (_S * ns), _S * ns)], idx_v)

    def do_rows(c0, tab_rows, n_rows):
        def desc(sc, buf):
            return pltpu.make_async_copy(
                stage_v.at[buf],
                gf_hbm.at[b, pl.ds(c0, n_rows), pl.ds(sc * _SCH, _SCH)],
                sem)

        def sc_loop(sc, _):
            buf = lax.rem(sc, 2)

            @pl.when(sc >= 2)
            def _drain():
                desc(sc, buf).wait()

            @plsc.parallel_loop(0, _SCH // 16, unroll=4)
            def g_loop(gi):
                off = sc * _SCH + gi * 16
                iv = idx_v[pl.ds(off, 16)]
                for r in range(n_rows):
                    rv = jnp.full((16,), r, jnp.int32)
                    stage_v[buf, r, pl.ds(gi * 16, 16)] = plsc.load_gather(
                        tab_rows, [rv, iv])
            desc(sc, buf).start()
            return 0
        lax.fori_loop(0, nsc, sc_loop, 0)
        for buf in range(2):
            desc(0, buf).wait()

    for sub in range(_CG // _SUB):
        c0 = g * _CG + sub * _SUB
        for r in range(_SUB):
            pltpu.sync_copy(
                feat_hbm.at[pl.ds((b * _C + c0 + r) * _N, _N)], tabs_v.at[r])
        do_rows(c0, tabs_v, _SUB)

    @pl.when(g < 3)
    def _xyz():
        pltpu.sync_copy(xyzt_hbm.at[pl.ds((b * 3 + g) * _N, _N)], xtab_v)
        pltpu.sync_copy(qt_hbm.at[pl.ds((b * 3 + g) * _S, _S)], q_v)

        def sc_loop(sc, _):
            @plsc.parallel_loop(0, _SCH // 16, unroll=4)
            def g_loop(gi):
                off = sc * _SCH + gi * 16
                iv = idx_v[pl.ds(off, 16)]
                vals = plsc.load_gather(xtab_v, [iv])
                si = lax.shift_right_logical(lax.iota(jnp.int32, 16) + off, 5)
                qs = plsc.load_gather(q_v, [si])
                xstage_v[pl.ds(gi * 16, 16)] = (vals - qs) / _RADIUS
            pltpu.async_copy(
                xstage_v, gx_hbm.at[b, g, pl.ds(sc * _SCH, _SCH)], sem
            ).wait()
            return 0
        lax.fori_loop(0, (_S * ns) // _SCH, sc_loop, 0)


def _sc_gather(key_features, idx, xyzt, qt):
    mesh = plsc.VectorSubcoreMesh(core_axis_name="c", subcore_axis_name="s")
    fn = functools.partial(
        pl.kernel, mesh=mesh,
        compiler_params=pltpu.CompilerParams(needs_layout_passes=False),
        out_type=[
            jax.ShapeDtypeStruct((_B, _C, _S * _NSAMPLE), jnp.float32),
            jax.ShapeDtypeStruct((_B, 3, _S * _NSAMPLE), jnp.float32),
        ],
        scratch_types=[
            pltpu.VMEM((_S * _NSAMPLE,), jnp.int32),
            pltpu.VMEM((_SUB, _N), jnp.float32),
            pltpu.VMEM((2, _SUB, _SCH), jnp.float32),
            pltpu.VMEM((_N,), jnp.float32),
            pltpu.VMEM((_SCH,), jnp.float32),
            pltpu.VMEM((_S,), jnp.float32),
            pltpu.SemaphoreType.DMA,
        ],
    )(_sc_gather_body)
    return fn(key_features, idx,
              xyzt.reshape(_B * 3 * _N), qt.reshape(_B * 3 * _S))


@jax.jit
def kernel(key_xyz, key_features, query_xyz):
    xyzt = jnp.transpose(key_xyz, (0, 2, 1))
    q = query_xyz[:, :, :3]
    qt = jnp.transpose(q, (0, 2, 1))

    nn = jnp.arange(_N, dtype=jnp.int32)
    pw = (1 << (nn % 16)).astype(jnp.float32)
    pow2 = jnp.where((nn[:, None] // 16)
                     == jnp.arange(_W, dtype=jnp.int32)[None, :],
                     pw[:, None], 0.0).astype(jnp.bfloat16)

    packed, featl = _ball_mask_packed(xyzt, q, pow2, key_features)
    idx = _sc_select(packed)
    gf, gx = _sc_gather(featl, idx, xyzt, qt)

    grouped_xyz = gx.reshape(_B, 3, _S, _NSAMPLE)
    grouped_features = gf.reshape(_B, _C, _S, _NSAMPLE)
    return grouped_xyz, grouped_features


# speedup vs baseline: 22.6231x; 1.4340x over previous
import functools

import jax
import jax.numpy as jnp
import numpy as np
from jax import lax
from jax.experimental import pallas as pl
from jax.experimental.pallas import tpu as pltpu
from jax.experimental.pallas import tpu_sc as plsc

_RADIUS = np.float32(0.2)
_RAD2 = np.float32(0.2 * 0.2)
_NSAMPLE = 32
_SBLK = 128
_B, _N, _S, _C = 8, 4096, 1024, 128
_W = _N // 16

_CG = _C // 4
_SUB = 8
_SCH = 2048

_QW = (_S * _B) // 32
_QCH = 16


def _mask_kernel(xyzt_ref, q_ref, pow2_ref, feat_ref, pk_ref, featl_ref):
    @pl.when(pl.program_id(1) == 0)
    def _pass_feat():
        for cc in range(_C):
            featl_ref[pl.ds(cc * _N, _N)] = feat_ref[0, cc]

    xyzt = xyzt_ref[0]
    q = q_ref[0]

    x2 = (xyzt[0:1, :] * xyzt[0:1, :]
          + xyzt[1:2, :] * xyzt[1:2, :]
          + xyzt[2:3, :] * xyzt[2:3, :])
    q2 = jnp.sum(q * q, axis=1, keepdims=True)
    qx = jnp.dot(q, xyzt, preferred_element_type=jnp.float32)
    d2 = q2 + x2 - 2.0 * qx
    mask = d2 < _RAD2

    packed = jnp.dot(mask.astype(jnp.bfloat16), pow2_ref[...],
                     preferred_element_type=jnp.float32)
    pk_ref[0] = packed.astype(jnp.int32)


def _ball_mask_packed(xyzt, q, pow2, feat):
    grid = (_B, _S // _SBLK)
    return pl.pallas_call(
        _mask_kernel,
        grid=grid,
        in_specs=[
            pl.BlockSpec((1, 3, _N), lambda i, j: (i, 0, 0)),
            pl.BlockSpec((1, _SBLK, 3), lambda i, j: (i, j, 0)),
            pl.BlockSpec((_N, _W), lambda i, j: (0, 0)),
            pl.BlockSpec((1, _C, _N), lambda i, j: (i, 0, 0)),
        ],
        out_specs=[
            pl.BlockSpec((1, _SBLK, _W), lambda i, j: (i, j, 0)),
            pl.BlockSpec((_C * _N,), lambda i, j: (i,)),
        ],
        out_shape=[
            jax.ShapeDtypeStruct((_B, _S, _W), jnp.int32),
            jax.ShapeDtypeStruct((_B * _C * _N,), jnp.float32),
        ],
    )(xyzt, q, pow2, feat)


def _swar_popcount16(x):
    x = x - (lax.shift_right_logical(x, 1) & 0x5555)
    x = (x & 0x3333) + (lax.shift_right_logical(x, 2) & 0x3333)
    x = (x + lax.shift_right_logical(x, 4)) & 0x0F0F
    return (x + lax.shift_right_logical(x, 8)) & 0x1F


def _sc_select_body(pk_hbm, idx_hbm, pw_v, wbuf_v, wibuf_v, cumbuf_v,
                    buf_v, out_v, sem):
    ns = _NSAMPLE
    wid = lax.axis_index("s") * 2 + lax.axis_index("c")
    q0 = wid * _QW
    lanes = lax.iota(jnp.int32, 16)

    def chunk_loop(ch, _):
        qbase = q0 + ch * _QCH
        pltpu.sync_copy(pk_hbm.at[pl.ds(qbase * _W, _QCH * _W)], pw_v)

        def q_loop(qi, _):
            def phase_a(t, carry):
                nzc, tot, kcnt = carry
                wv = pw_v[pl.ds(qi * _W + t * 16, 16)]
                nz = wv != 0
                pc = _swar_popcount16(wv)
                cumi = plsc.cumsum(pc) + tot
                cume = cumi - pc
                plsc.store_compressed(wbuf_v.at[pl.ds(nzc, 16)], wv, mask=nz)
                plsc.store_compressed(wibuf_v.at[pl.ds(nzc, 16)],
                                      lanes + t * 16, mask=nz)
                plsc.store_compressed(cumbuf_v.at[pl.ds(nzc, 16)], cume,
                                      mask=nz)
                nzc = nzc + plsc.all_reduce_population_count(nz)[0]
                kcnt = kcnt + plsc.all_reduce_population_count(
                    nz & (cume < ns))[0]
                return nzc, cumi[15], kcnt

            _, cnt, kw = lax.fori_loop(
                0, _W // 16, phase_a,
                (jnp.int32(0), jnp.int32(0), jnp.int32(0)))

            def phase_b(wi, _):
                wsp = jnp.zeros((16,), jnp.int32) + wi
                wvs = plsc.load_gather(wbuf_v, [wsp])
                wis = plsc.load_gather(wibuf_v, [wsp])
                cms = plsc.load_gather(cumbuf_v, [wsp])
                m = (lax.shift_right_logical(wvs, lanes) & 1) != 0
                m = m & (wi < kw)
                ivec = wis * 16 + lanes
                off = jnp.minimum(cms[0], 48)
                plsc.store_compressed(buf_v.at[pl.ds(off, 16)], ivec, mask=m)
                return 0
            lax.fori_loop(0, ns, phase_b, 0)

            v0 = buf_v[pl.ds(0, 16)]
            padv = jnp.where(cnt > 0, jnp.zeros((16,), jnp.int32) + v0[0], 0)
            for h in range(2):
                vals = buf_v[pl.ds(h * 16, 16)]
                pos = lanes + h * 16
                out_v[pl.ds(qi * ns + h * 16, 16)] = jnp.where(
                    pos < cnt, vals, padv)
            return 0

        lax.fori_loop(0, _QCH, q_loop, 0)
        pltpu.async_copy(
            out_v, idx_hbm.at[pl.ds(qbase * ns, _QCH * ns)], sem).wait()
        return 0

    lax.fori_loop(0, _QW // _QCH, chunk_loop, 0)


def _sc_select(packed):
    mesh = plsc.VectorSubcoreMesh(core_axis_name="c", subcore_axis_name="s")
    fn = functools.partial(
        pl.kernel, mesh=mesh,
        compiler_params=pltpu.CompilerParams(needs_layout_passes=False),
        out_type=jax.ShapeDtypeStruct((_B * _S * _NSAMPLE,), jnp.int32),
        scratch_types=[
            pltpu.VMEM((_QCH * _W,), jnp.int32),
            pltpu.VMEM((288,), jnp.int32),
            pltpu.VMEM((288,), jnp.int32),
            pltpu.VMEM((288,), jnp.int32),
            pltpu.VMEM((64,), jnp.int32),
            pltpu.VMEM((_QCH * _NSAMPLE,), jnp.int32),
            pltpu.SemaphoreType.DMA,
        ],
    )(_sc_select_body)
    return fn(packed.reshape(_B * _S * _W))


def _sc_gather_body(feat_hbm, idx_hbm, xyzt_hbm, qt_hbm,
                    gf_hbm, gx_hbm,
                    idx_v, tabs_v, stage_v, xtab_v, xstage_v, q_v, sem):
    ns = _NSAMPLE
    nsc = (_S * ns) // _SCH
    wid = lax.axis_index("s") * 2 + lax.axis_index("c")
    b = wid // 4
    g = wid % 4

    pltpu.sync_copy(idx_hbm.at[pl.ds(b * (_S * ns), _S * ns)], idx_v)

    def do_rows(c0, tab_rows, n_rows):
        def desc(sc, buf):
            return pltpu.make_async_copy(
                stage_v.at[buf],
                gf_hbm.at[b, pl.ds(c0, n_rows), pl.ds(sc * _SCH, _SCH)],
                sem)

        def sc_loop(sc, _):
            buf = lax.rem(sc, 2)

            @pl.when(sc >= 2)
            def _drain():
                desc(sc, buf).wait()

            @plsc.parallel_loop(0, _SCH // 16, unroll=4)
            def g_loop(gi):
                off = sc * _SCH + gi * 16
                iv = idx_v[pl.ds(off, 16)]
                for r in range(n_rows):
                    rv = jnp.full((16,), r, jnp.int32)
                    stage_v[buf, r, pl.ds(gi * 16, 16)] = plsc.load_gather(
                        tab_rows, [rv, iv])
            desc(sc, buf).start()
            return 0
        lax.fori_loop(0, nsc, sc_loop, 0)
        for buf in range(2):
            desc(0, buf).wait()

    for sub in range(_CG // _SUB):
        c0 = g * _CG + sub * _SUB
        for r in range(_SUB):
            pltpu.sync_copy(
                feat_hbm.at[pl.ds((b * _C + c0 + r) * _N, _N)], tabs_v.at[r])
        do_rows(c0, tabs_v, _SUB)

    @pl.when(g < 3)
    def _xyz():
        pltpu.sync_copy(xyzt_hbm.at[pl.ds((b * 3 + g) * _N, _N)], xtab_v)
        pltpu.sync_copy(qt_hbm.at[pl.ds((b * 3 + g) * _S, _S)], q_v)

        def sc_loop(sc, _):
            @plsc.parallel_loop(0, _SCH // 16, unroll=4)
            def g_loop(gi):
                off = sc * _SCH + gi * 16
                iv = idx_v[pl.ds(off, 16)]
                vals = plsc.load_gather(xtab_v, [iv])
                si = lax.shift_right_logical(lax.iota(jnp.int32, 16) + off, 5)
                qs = plsc.load_gather(q_v, [si])
                xstage_v[pl.ds(gi * 16, 16)] = (vals - qs) / _RADIUS
            pltpu.async_copy(
                xstage_v, gx_hbm.at[b, g, pl.ds(sc * _SCH, _SCH)], sem
            ).wait()
            return 0
        lax.fori_loop(0, (_S * ns) // _SCH, sc_loop, 0)


def _sc_gather(key_features, idx, xyzt, qt):
    mesh = plsc.VectorSubcoreMesh(core_axis_name="c", subcore_axis_name="s")
    fn = functools.partial(
        pl.kernel, mesh=mesh,
        compiler_params=pltpu.CompilerParams(needs_layout_passes=False),
        out_type=[
            jax.ShapeDtypeStruct((_B, _C, _S * _NSAMPLE), jnp.float32),
            jax.ShapeDtypeStruct((_B, 3, _S * _NSAMPLE), jnp.float32),
        ],
        scratch_types=[
            pltpu.VMEM((_S * _NSAMPLE,), jnp.int32),
            pltpu.VMEM((_SUB, _N), jnp.float32),
            pltpu.VMEM((2, _SUB, _SCH), jnp.float32),
            pltpu.VMEM((_N,), jnp.float32),
            pltpu.VMEM((_SCH,), jnp.float32),
            pltpu.VMEM((_S,), jnp.float32),
            pltpu.SemaphoreType.DMA,
        ],
    )(_sc_gather_body)
    return fn(key_features, idx,
              xyzt.reshape(_B * 3 * _N), qt.reshape(_B * 3 * _S))


@jax.jit
def kernel(key_xyz, key_features, query_xyz):
    xyzt = jnp.transpose(key_xyz, (0, 2, 1))
    q = query_xyz[:, :, :3]
    qt = jnp.transpose(q, (0, 2, 1))

    nn = jnp.arange(_N, dtype=jnp.int32)
    pw = (1 << (nn % 16)).astype(jnp.float32)
    pow2 = jnp.where((nn[:, None] // 16)
                     == jnp.arange(_W, dtype=jnp.int32)[None, :],
                     pw[:, None], 0.0).astype(jnp.bfloat16)

    packed, featl = _ball_mask_packed(xyzt, q, pow2, key_features)
    idx = _sc_select(packed)
    gf, gx = _sc_gather(featl, idx, xyzt, qt)

    grouped_xyz = gx.reshape(_B, 3, _S, _NSAMPLE)
    grouped_features = gf.reshape(_B, _C, _S, _NSAMPLE)
    return grouped_xyz, grouped_features


# speedup vs baseline: 27.4998x; 1.2156x over previous
import functools

import jax
import jax.numpy as jnp
import numpy as np
from jax import lax
from jax.experimental import pallas as pl
from jax.experimental.pallas import tpu as pltpu
from jax.experimental.pallas import tpu_sc as plsc

_RADIUS = np.float32(0.2)
_RAD2 = np.float32(0.2 * 0.2)
_NSAMPLE = 32
_SBLK = 128
_B, _N, _S, _C = 8, 4096, 1024, 128
_W = _N // 16

_CG = _C // 4
_SUB = 8
_SCH = 2048

_QW = (_S * _B) // 32
_QCH = 16


def _mask_kernel(xyzt_ref, q_ref, pow2_ref, feat_ref, pk_ref, featl_ref):
    @pl.when(pl.program_id(1) == 0)
    def _pass_feat():
        for cc in range(_C):
            featl_ref[pl.ds(cc * _N, _N)] = feat_ref[0, cc]

    xyzt = xyzt_ref[0]
    q = q_ref[0]

    x2 = (xyzt[0:1, :] * xyzt[0:1, :]
          + xyzt[1:2, :] * xyzt[1:2, :]
          + xyzt[2:3, :] * xyzt[2:3, :])
    q2 = jnp.sum(q * q, axis=1, keepdims=True)
    qx = jnp.dot(q, xyzt, preferred_element_type=jnp.float32)
    d2 = q2 + x2 - 2.0 * qx
    mask = d2 < _RAD2

    packed = jnp.dot(mask.astype(jnp.bfloat16), pow2_ref[...],
                     preferred_element_type=jnp.float32)
    pk_ref[0] = packed.astype(jnp.int32)


def _ball_mask_packed(xyzt, q, pow2, feat):
    grid = (_B, _S // _SBLK)
    return pl.pallas_call(
        _mask_kernel,
        grid=grid,
        in_specs=[
            pl.BlockSpec((1, 3, _N), lambda i, j: (i, 0, 0)),
            pl.BlockSpec((1, _SBLK, 3), lambda i, j: (i, j, 0)),
            pl.BlockSpec((_N, _W), lambda i, j: (0, 0)),
            pl.BlockSpec((1, _C, _N), lambda i, j: (i, 0, 0)),
        ],
        out_specs=[
            pl.BlockSpec((1, _SBLK, _W), lambda i, j: (i, j, 0)),
            pl.BlockSpec((_C * _N,), lambda i, j: (i,)),
        ],
        out_shape=[
            jax.ShapeDtypeStruct((_B, _S, _W), jnp.int32),
            jax.ShapeDtypeStruct((_B * _C * _N,), jnp.float32),
        ],
    )(xyzt, q, pow2, feat)


def _swar_popcount16(x):
    x = x - (lax.shift_right_logical(x, 1) & 0x5555)
    x = (x & 0x3333) + (lax.shift_right_logical(x, 2) & 0x3333)
    x = (x + lax.shift_right_logical(x, 4)) & 0x0F0F
    return (x + lax.shift_right_logical(x, 8)) & 0x1F


def _sc_select_body(pk_hbm, idx_hbm, pw_v, wbuf_v, wibuf_v, cumbuf_v,
                    buf_v, out_v, sem):
    ns = _NSAMPLE
    wid = lax.axis_index("s") * 2 + lax.axis_index("c")
    q0 = wid * _QW
    lanes = lax.iota(jnp.int32, 16)

    def chunk_loop(ch, _):
        qbase = q0 + ch * _QCH
        pltpu.sync_copy(pk_hbm.at[pl.ds(qbase * _W, _QCH * _W)], pw_v)

        def q_loop(qi, _):
            def phase_a(t, carry):
                nzc, tot, kcnt = carry
                wv = pw_v[pl.ds(qi * _W + t * 16, 16)]
                nz = wv != 0
                pc = _swar_popcount16(wv)
                cumi = plsc.cumsum(pc) + tot
                cume = cumi - pc
                plsc.store_compressed(wbuf_v.at[pl.ds(nzc, 16)], wv, mask=nz)
                plsc.store_compressed(wibuf_v.at[pl.ds(nzc, 16)],
                                      lanes + t * 16, mask=nz)
                plsc.store_compressed(cumbuf_v.at[pl.ds(nzc, 16)], cume,
                                      mask=nz)
                nzc = nzc + plsc.all_reduce_population_count(nz)[0]
                kcnt = kcnt + plsc.all_reduce_population_count(
                    nz & (cume < ns))[0]
                return nzc, cumi[15], kcnt

            _, cnt, kw = lax.fori_loop(
                0, _W // 16, phase_a,
                (jnp.int32(0), jnp.int32(0), jnp.int32(0)))

            @plsc.parallel_loop(0, ns, unroll=4)
            def phase_b(wi):
                wsp = jnp.zeros((16,), jnp.int32) + wi
                wvs = plsc.load_gather(wbuf_v, [wsp])
                wis = plsc.load_gather(wibuf_v, [wsp])
                cms = plsc.load_gather(cumbuf_v, [wsp])
                m = (lax.shift_right_logical(wvs, lanes) & 1) != 0
                m = m & (wi < kw)
                ivec = wis * 16 + lanes
                off = jnp.minimum(cms[0], 48)
                plsc.store_compressed(buf_v.at[pl.ds(off, 16)], ivec, mask=m)

            v0 = buf_v[pl.ds(0, 16)]
            padv = jnp.where(cnt > 0, jnp.zeros((16,), jnp.int32) + v0[0], 0)
            for h in range(2):
                vals = buf_v[pl.ds(h * 16, 16)]
                pos = lanes + h * 16
                out_v[pl.ds(qi * ns + h * 16, 16)] = jnp.where(
                    pos < cnt, vals, padv)
            return 0

        lax.fori_loop(0, _QCH, q_loop, 0)
        pltpu.async_copy(
            out_v, idx_hbm.at[pl.ds(qbase * ns, _QCH * ns)], sem).wait()
        return 0

    lax.fori_loop(0, _QW // _QCH, chunk_loop, 0)


def _sc_select(packed):
    mesh = plsc.VectorSubcoreMesh(core_axis_name="c", subcore_axis_name="s")
    fn = functools.partial(
        pl.kernel, mesh=mesh,
        compiler_params=pltpu.CompilerParams(needs_layout_passes=False),
        out_type=jax.ShapeDtypeStruct((_B * _S * _NSAMPLE,), jnp.int32),
        scratch_types=[
            pltpu.VMEM((_QCH * _W,), jnp.int32),
            pltpu.VMEM((288,), jnp.int32),
            pltpu.VMEM((288,), jnp.int32),
            pltpu.VMEM((288,), jnp.int32),
            pltpu.VMEM((64,), jnp.int32),
            pltpu.VMEM((_QCH * _NSAMPLE,), jnp.int32),
            pltpu.SemaphoreType.DMA,
        ],
    )(_sc_select_body)
    return fn(packed.reshape(_B * _S * _W))


def _sc_gather_body(feat_hbm, idx_hbm, xyzt_hbm, qt_hbm,
                    gf_hbm, gx_hbm,
                    idx_v, tabs_v, stage_v, xtab_v, xstage_v, q_v, sem):
    ns = _NSAMPLE
    nsc = (_S * ns) // _SCH
    wid = lax.axis_index("s") * 2 + lax.axis_index("c")
    b = wid // 4
    g = wid % 4

    pltpu.sync_copy(idx_hbm.at[pl.ds(b * (_S * ns), _S * ns)], idx_v)

    def do_rows(c0, tab_rows, n_rows):
        def desc(sc, buf):
            return pltpu.make_async_copy(
                stage_v.at[buf],
                gf_hbm.at[b, pl.ds(c0, n_rows), pl.ds(sc * _SCH, _SCH)],
                sem)

        def sc_loop(sc, _):
            buf = lax.rem(sc, 2)

            @pl.when(sc >= 2)
            def _drain():
                desc(sc, buf).wait()

            @plsc.parallel_loop(0, _SCH // 16, unroll=4)
            def g_loop(gi):
                off = sc * _SCH + gi * 16
                iv = idx_v[pl.ds(off, 16)]
                for r in range(n_rows):
                    rv = jnp.full((16,), r, jnp.int32)
                    stage_v[buf, r, pl.ds(gi * 16, 16)] = plsc.load_gather(
                        tab_rows, [rv, iv])
            desc(sc, buf).start()
            return 0
        lax.fori_loop(0, nsc, sc_loop, 0)
        for buf in range(2):
            desc(0, buf).wait()

    for sub in range(_CG // _SUB):
        c0 = g * _CG + sub * _SUB
        for r in range(_SUB):
            pltpu.sync_copy(
                feat_hbm.at[pl.ds((b * _C + c0 + r) * _N, _N)], tabs_v.at[r])
        do_rows(c0, tabs_v, _SUB)

    @pl.when(g < 3)
    def _xyz():
        pltpu.sync_copy(xyzt_hbm.at[pl.ds((b * 3 + g) * _N, _N)], xtab_v)
        pltpu.sync_copy(qt_hbm.at[pl.ds((b * 3 + g) * _S, _S)], q_v)

        def sc_loop(sc, _):
            @plsc.parallel_loop(0, _SCH // 16, unroll=4)
            def g_loop(gi):
                off = sc * _SCH + gi * 16
                iv = idx_v[pl.ds(off, 16)]
                vals = plsc.load_gather(xtab_v, [iv])
                si = lax.shift_right_logical(lax.iota(jnp.int32, 16) + off, 5)
                qs = plsc.load_gather(q_v, [si])
                xstage_v[pl.ds(gi * 16, 16)] = (vals - qs) / _RADIUS
            pltpu.async_copy(
                xstage_v, gx_hbm.at[b, g, pl.ds(sc * _SCH, _SCH)], sem
            ).wait()
            return 0
        lax.fori_loop(0, (_S * ns) // _SCH, sc_loop, 0)


def _sc_gather(key_features, idx, xyzt, qt):
    mesh = plsc.VectorSubcoreMesh(core_axis_name="c", subcore_axis_name="s")
    fn = functools.partial(
        pl.kernel, mesh=mesh,
        compiler_params=pltpu.CompilerParams(needs_layout_passes=False),
        out_type=[
            jax.ShapeDtypeStruct((_B, _C, _S * _NSAMPLE), jnp.float32),
            jax.ShapeDtypeStruct((_B, 3, _S * _NSAMPLE), jnp.float32),
        ],
        scratch_types=[
            pltpu.VMEM((_S * _NSAMPLE,), jnp.int32),
            pltpu.VMEM((_SUB, _N), jnp.float32),
            pltpu.VMEM((2, _SUB, _SCH), jnp.float32),
            pltpu.VMEM((_N,), jnp.float32),
            pltpu.VMEM((_SCH,), jnp.float32),
            pltpu.VMEM((_S,), jnp.float32),
            pltpu.SemaphoreType.DMA,
        ],
    )(_sc_gather_body)
    return fn(key_features, idx,
              xyzt.reshape(_B * 3 * _N), qt.reshape(_B * 3 * _S))


@jax.jit
def kernel(key_xyz, key_features, query_xyz):
    xyzt = jnp.transpose(key_xyz, (0, 2, 1))
    q = query_xyz[:, :, :3]
    qt = jnp.transpose(q, (0, 2, 1))

    nn = jnp.arange(_N, dtype=jnp.int32)
    pw = (1 << (nn % 16)).astype(jnp.float32)
    pow2 = jnp.where((nn[:, None] // 16)
                     == jnp.arange(_W, dtype=jnp.int32)[None, :],
                     pw[:, None], 0.0).astype(jnp.bfloat16)

    packed, featl = _ball_mask_packed(xyzt, q, pow2, key_features)
    idx = _sc_select(packed)
    gf, gx = _sc_gather(featl, idx, xyzt, qt)

    grouped_xyz = gx.reshape(_B, 3, _S, _NSAMPLE)
    grouped_features = gf.reshape(_B, _C, _S, _NSAMPLE)
    return grouped_xyz, grouped_features
